# baseline (speedup 1.0000x reference)
import functools

import jax
import jax.numpy as jnp
from jax import lax
from jax.experimental import pallas as pl
from jax.experimental.pallas import tpu as pltpu
from jax.experimental.pallas import tpu_sc as plsc

B, C, N, OUTC, K = 8, 256, 2048, 256, 1024


def _nf_body(x_ref, gamma_ref, beta_ref, w1_ref, b1_ref, w2_ref, b2_ref,
             nf_ref):
    x = x_ref[0]
    mu = jnp.mean(x, axis=0, keepdims=True)
    var = jnp.mean((x - mu) * (x - mu), axis=0, keepdims=True)
    fT = (x - mu) / jnp.sqrt(var + 1e-6) * gamma_ref[...] + beta_ref[...]
    hT = jax.lax.dot_general(w1_ref[...], fT, (((0,), (0,)), ((), ())),
                             preferred_element_type=jnp.float32)
    hT = jnp.maximum(hT + b1_ref[...], 0.0)
    nf = jax.lax.dot_general(hT, w2_ref[...], (((0,), (0,)), ((), ())),
                             preferred_element_type=jnp.float32)
    nf_ref[0] = nf + b2_ref[...]


def _nf_pallas(features, gamma, beta, W1, b1, W2, b2):
    return pl.pallas_call(
        _nf_body,
        grid=(B,),
        in_specs=[
            pl.BlockSpec((1, C, N), lambda b: (b, 0, 0)),
            pl.BlockSpec((C, 1), lambda b: (0, 0)),
            pl.BlockSpec((C, 1), lambda b: (0, 0)),
            pl.BlockSpec((C, C), lambda b: (0, 0)),
            pl.BlockSpec((C, 1), lambda b: (0, 0)),
            pl.BlockSpec((C, OUTC), lambda b: (0, 0)),
            pl.BlockSpec((1, OUTC), lambda b: (0, 0)),
        ],
        out_specs=pl.BlockSpec((1, N, OUTC), lambda b: (b, 0, 0)),
        out_shape=jax.ShapeDtypeStruct((B, N, OUTC), jnp.float32),
    )(features, gamma[:, None], beta[:, None], W1, b1[:, None], W2, b2[None, :])


def _sort_body(y_ref, idx_ref):
    v = y_ref[...]
    idx = lax.broadcasted_iota(jnp.int32, (B, N), 1)
    pos = lax.broadcasted_iota(jnp.int32, (B, N), 1)
    k = 2
    while k <= N:
        j = k // 2
        while j >= 1:
            mask_lo = (pos & j) == 0
            pv = jnp.where(mask_lo, jnp.roll(v, -j, axis=1),
                           jnp.roll(v, j, axis=1))
            pidx = jnp.where(mask_lo, jnp.roll(idx, -j, axis=1),
                             jnp.roll(idx, j, axis=1))
            take_max = ((pos & k) == 0) == mask_lo
            self_wins = (v > pv) | ((v == pv) & (idx < pidx))
            keep_self = take_max == self_wins
            v = jnp.where(keep_self, v, pv)
            idx = jnp.where(keep_self, idx, pidx)
            j //= 2
        k *= 2
    idx_ref[...] = idx[:, :K]


def _sort_pallas(y):
    return pl.pallas_call(
        _sort_body,
        out_shape=jax.ShapeDtypeStruct((B, K), jnp.int32),
    )(y)


_NW = 32
_RPW = (B * K) // _NW
_CHUNK = 128


def _sc_gather_body(nf_hbm, xyz_hbm, gidx_hbm, feats_out, xyz_out,
                    idx_v, rows_v, xrows_v, sem):
    wid = lax.axis_index("s") * 2 + lax.axis_index("c")
    base = wid * _RPW
    pltpu.sync_copy(gidx_hbm.at[pl.ds(base, _RPW)], idx_v)
    copies = []
    for j in range(_RPW // _CHUNK):
        sl = pl.ds(j * _CHUNK, _CHUNK)
        copies.append(pltpu.async_copy(nf_hbm.at[idx_v.at[sl]],
                                       rows_v.at[sl], sem))
        copies.append(pltpu.async_copy(xyz_hbm.at[idx_v.at[sl]],
                                       xrows_v.at[sl], sem))
    for cp in copies:
        cp.wait()
    pltpu.sync_copy(rows_v, feats_out.at[pl.ds(base, _RPW)])
    pltpu.sync_copy(xrows_v, xyz_out.at[pl.ds(base, _RPW)])


def _sc_gather(nf_flat, xyz_flat, gidx):
    mesh = plsc.VectorSubcoreMesh(core_axis_name="c", subcore_axis_name="s")
    fn = functools.partial(
        pl.kernel, mesh=mesh,
        out_type=(jax.ShapeDtypeStruct((B * K, OUTC), jnp.float32),
                  jax.ShapeDtypeStruct((B * K, 16), jnp.float32)),
        scratch_types=[
            pltpu.VMEM((_RPW,), jnp.int32),
            pltpu.VMEM((_RPW, OUTC), jnp.float32),
            pltpu.VMEM((_RPW, 16), jnp.float32),
            pltpu.SemaphoreType.DMA,
        ],
        compiler_params=pltpu.CompilerParams(use_tc_tiling_on_sc=False),
    )(_sc_gather_body)
    return fn(nf_flat, xyz_flat, gidx)


def kernel(xyzs, features, gamma, beta, W1, b1, W2, b2, W3, b3, W4, b4):
    f = jnp.transpose(features, (0, 2, 1))
    mu = jnp.mean(f, axis=-1, keepdims=True)
    var = jnp.var(f, axis=-1, keepdims=True)
    f = (f - mu) / jnp.sqrt(var + 1e-6) * gamma + beta
    g = jax.nn.relu(f @ W3 + b3)
    w = (g @ W4 + b4)[:, :, 0]
    y = w / 0.1

    nf = _nf_pallas(features, gamma, beta, W1, b1, W2, b2)
    indices = _sort_pallas(y)

    gidx = (indices + (jnp.arange(B, dtype=jnp.int32) * N)[:, None]).reshape(-1)
    nf_flat = nf.reshape(B * N, OUTC)
    xyz_flat = jnp.concatenate(
        [xyzs, jnp.zeros((B, N, 13), jnp.float32)], axis=2).reshape(B * N, 16)

    feats_g, xyz_g = _sc_gather(nf_flat, xyz_flat, gidx)
    feats_out = jnp.transpose(feats_g.reshape(B, K, OUTC), (0, 2, 1))
    xyzs_out = xyz_g.reshape(B, K, 16)[:, :, :3]
    return (xyzs_out, feats_out, indices)


# speedup vs baseline: 1.1945x; 1.1945x over previous
import functools

import jax
import jax.numpy as jnp
from jax import lax
from jax.experimental import pallas as pl
from jax.experimental.pallas import tpu as pltpu
from jax.experimental.pallas import tpu_sc as plsc

B, C, N, OUTC, K = 8, 256, 2048, 256, 1024


def _nf_body(x_ref, gamma_ref, beta_ref, w1_ref, b1_ref, w2_ref, b2_ref,
             nf_ref):
    x = x_ref[0]
    mu = jnp.mean(x, axis=0, keepdims=True)
    var = jnp.mean((x - mu) * (x - mu), axis=0, keepdims=True)
    fT = (x - mu) / jnp.sqrt(var + 1e-6) * gamma_ref[...] + beta_ref[...]
    hT = jax.lax.dot_general(w1_ref[...], fT, (((0,), (0,)), ((), ())),
                             preferred_element_type=jnp.float32)
    hT = jnp.maximum(hT + b1_ref[...], 0.0)
    nf = jax.lax.dot_general(hT, w2_ref[...], (((0,), (0,)), ((), ())),
                             preferred_element_type=jnp.float32)
    nf_ref[0] = nf + b2_ref[...]


def _nf_pallas(features, gamma, beta, W1, b1, W2, b2):
    return pl.pallas_call(
        _nf_body,
        grid=(B,),
        in_specs=[
            pl.BlockSpec((1, C, N), lambda b: (b, 0, 0)),
            pl.BlockSpec((C, 1), lambda b: (0, 0)),
            pl.BlockSpec((C, 1), lambda b: (0, 0)),
            pl.BlockSpec((C, C), lambda b: (0, 0)),
            pl.BlockSpec((C, 1), lambda b: (0, 0)),
            pl.BlockSpec((C, OUTC), lambda b: (0, 0)),
            pl.BlockSpec((1, OUTC), lambda b: (0, 0)),
        ],
        out_specs=pl.BlockSpec((1, N, OUTC), lambda b: (b, 0, 0)),
        out_shape=jax.ShapeDtypeStruct((B, N, OUTC), jnp.float32),
    )(features, gamma[:, None], beta[:, None], W1, b1[:, None], W2, b2[None, :])


def _sort_body(y_ref, idx_ref):
    v = y_ref[...]
    idx = lax.broadcasted_iota(jnp.int32, (B, N), 1)
    pos = lax.broadcasted_iota(jnp.int32, (B, N), 1)
    k = 2
    while k <= N:
        j = k // 2
        while j >= 1:
            mask_lo = (pos & j) == 0
            pv = jnp.where(mask_lo, jnp.roll(v, -j, axis=1),
                           jnp.roll(v, j, axis=1))
            pidx = jnp.where(mask_lo, jnp.roll(idx, -j, axis=1),
                             jnp.roll(idx, j, axis=1))
            take_max = ((pos & k) == 0) == mask_lo
            self_wins = (v > pv) | ((v == pv) & (idx < pidx))
            keep_self = take_max == self_wins
            v = jnp.where(keep_self, v, pv)
            idx = jnp.where(keep_self, idx, pidx)
            j //= 2
        k *= 2
    idx_ref[...] = idx[:, :K]


def _sort_pallas(y):
    return pl.pallas_call(
        _sort_body,
        out_shape=jax.ShapeDtypeStruct((B, K), jnp.int32),
    )(y)


_NW = 32
_RPW = (B * K) // _NW
_CHUNK = 128


def _sc_gather_body(nf_hbm, xyz_hbm, gidx_hbm, feats_out, xyz_out,
                    idx_v, rows_v, xrows_v, sem):
    wid = lax.axis_index("s") * 2 + lax.axis_index("c")
    base = wid * _RPW
    pltpu.sync_copy(gidx_hbm.at[pl.ds(base, _RPW)], idx_v)
    copies = []
    for j in range(_RPW // _CHUNK):
        sl = pl.ds(j * _CHUNK, _CHUNK)
        copies.append(pltpu.async_copy(nf_hbm.at[idx_v.at[sl]],
                                       rows_v.at[sl], sem))
        copies.append(pltpu.async_copy(xyz_hbm.at[idx_v.at[sl]],
                                       xrows_v.at[sl], sem))
    for cp in copies:
        cp.wait()
    pltpu.sync_copy(rows_v, feats_out.at[pl.ds(base, _RPW)])
    pltpu.sync_copy(xrows_v, xyz_out.at[pl.ds(base, _RPW)])


def _sc_gather(nf_flat, xyz_flat, gidx):
    mesh = plsc.VectorSubcoreMesh(core_axis_name="c", subcore_axis_name="s")
    fn = functools.partial(
        pl.kernel, mesh=mesh,
        out_type=(jax.ShapeDtypeStruct((B * K, OUTC), jnp.float32),
                  jax.ShapeDtypeStruct((B * K, 128), jnp.float32)),
        scratch_types=[
            pltpu.VMEM((_RPW,), jnp.int32),
            pltpu.VMEM((_RPW, OUTC), jnp.float32),
            pltpu.VMEM((_RPW, 128), jnp.float32),
            pltpu.SemaphoreType.DMA,
        ],
    )(_sc_gather_body)
    return fn(nf_flat, xyz_flat, gidx)


def kernel(xyzs, features, gamma, beta, W1, b1, W2, b2, W3, b3, W4, b4):
    f = jnp.transpose(features, (0, 2, 1))
    mu = jnp.mean(f, axis=-1, keepdims=True)
    var = jnp.var(f, axis=-1, keepdims=True)
    f = (f - mu) / jnp.sqrt(var + 1e-6) * gamma + beta
    g = jax.nn.relu(f @ W3 + b3)
    w = (g @ W4 + b4)[:, :, 0]
    y = w / 0.1

    nf = _nf_pallas(features, gamma, beta, W1, b1, W2, b2)
    indices = _sort_pallas(y)

    gidx = (indices + (jnp.arange(B, dtype=jnp.int32) * N)[:, None]).reshape(-1)
    nf_flat = nf.reshape(B * N, OUTC)
    xyz_flat = jnp.concatenate(
        [xyzs, jnp.zeros((B, N, 125), jnp.float32)], axis=2).reshape(B * N, 128)

    feats_g, xyz_g = _sc_gather(nf_flat, xyz_flat, gidx)
    feats_out = jnp.transpose(feats_g.reshape(B, K, OUTC), (0, 2, 1))
    xyzs_out = xyz_g.reshape(B, K, 128)[:, :, :3]
    return (xyzs_out, feats_out, indices)


# speedup vs baseline: 1.2346x; 1.0336x over previous
import functools

import jax
import jax.numpy as jnp
from jax import lax
from jax.experimental import pallas as pl
from jax.experimental.pallas import tpu as pltpu
from jax.experimental.pallas import tpu_sc as plsc

B, C, N, OUTC, K = 8, 256, 2048, 256, 1024


def _nf_body(x_ref, xyz_ref, gamma_ref, beta_ref, w1_ref, b1_ref, w2_ref,
             b2_ref, nf_ref, xyzpad_ref):
    xyzpad_ref[0] = jnp.concatenate(
        [xyz_ref[0], jnp.zeros((N, 125), jnp.float32)], axis=1)
    x = x_ref[0]
    mu = jnp.mean(x, axis=0, keepdims=True)
    var = jnp.mean((x - mu) * (x - mu), axis=0, keepdims=True)
    fT = (x - mu) / jnp.sqrt(var + 1e-6) * gamma_ref[...] + beta_ref[...]
    hT = jax.lax.dot_general(w1_ref[...], fT, (((0,), (0,)), ((), ())),
                             preferred_element_type=jnp.float32)
    hT = jnp.maximum(hT + b1_ref[...], 0.0)
    nf = jax.lax.dot_general(hT, w2_ref[...], (((0,), (0,)), ((), ())),
                             preferred_element_type=jnp.float32)
    nf_ref[0] = nf + b2_ref[...]


def _nf_pallas(features, xyzs, gamma, beta, W1, b1, W2, b2):
    return pl.pallas_call(
        _nf_body,
        grid=(B,),
        in_specs=[
            pl.BlockSpec((1, C, N), lambda b: (b, 0, 0)),
            pl.BlockSpec((1, N, 3), lambda b: (b, 0, 0)),
            pl.BlockSpec((C, 1), lambda b: (0, 0)),
            pl.BlockSpec((C, 1), lambda b: (0, 0)),
            pl.BlockSpec((C, C), lambda b: (0, 0)),
            pl.BlockSpec((C, 1), lambda b: (0, 0)),
            pl.BlockSpec((C, OUTC), lambda b: (0, 0)),
            pl.BlockSpec((1, OUTC), lambda b: (0, 0)),
        ],
        out_specs=[
            pl.BlockSpec((1, N, OUTC), lambda b: (b, 0, 0)),
            pl.BlockSpec((1, N, 128), lambda b: (b, 0, 0)),
        ],
        out_shape=[
            jax.ShapeDtypeStruct((B, N, OUTC), jnp.float32),
            jax.ShapeDtypeStruct((B, N, 128), jnp.float32),
        ],
    )(features, xyzs, gamma[:, None], beta[:, None], W1, b1[:, None], W2,
      b2[None, :])


def _sort_body(y_ref, idx_ref, gidx_ref):
    v = y_ref[...]
    idx = lax.broadcasted_iota(jnp.int32, (B, N), 1)
    pos = lax.broadcasted_iota(jnp.int32, (B, N), 1)
    k = 2
    while k <= N:
        j = k // 2
        while j >= 1:
            mask_lo = (pos & j) == 0
            pv = jnp.where(mask_lo, jnp.roll(v, -j, axis=1),
                           jnp.roll(v, j, axis=1))
            pidx = jnp.where(mask_lo, jnp.roll(idx, -j, axis=1),
                             jnp.roll(idx, j, axis=1))
            take_max = ((pos & k) == 0) == mask_lo
            self_wins = (v > pv) | ((v == pv) & (idx < pidx))
            keep_self = take_max == self_wins
            v = jnp.where(keep_self, v, pv)
            idx = jnp.where(keep_self, idx, pidx)
            j //= 2
        k *= 2
    topk = idx[:, :K]
    idx_ref[...] = topk
    gidx_ref[...] = topk + lax.broadcasted_iota(jnp.int32, (B, K), 0) * N


def _sort_pallas(y):
    return pl.pallas_call(
        _sort_body,
        out_shape=[jax.ShapeDtypeStruct((B, K), jnp.int32),
                   jax.ShapeDtypeStruct((B, K), jnp.int32)],
    )(y)


_NW = 32
_RPW = (B * K) // _NW
_CHUNK = 128


def _sc_gather_body(nf_hbm, xyz_hbm, gidx_hbm, feats_out, xyz_out,
                    idx_v, rows_v, xrows_v, sem):
    wid = lax.axis_index("s") * 2 + lax.axis_index("c")
    base = wid * _RPW
    pltpu.sync_copy(gidx_hbm.at[pl.ds(base, _RPW)], idx_v)
    copies = []
    for j in range(_RPW // _CHUNK):
        sl = pl.ds(j * _CHUNK, _CHUNK)
        copies.append(pltpu.async_copy(nf_hbm.at[idx_v.at[sl]],
                                       rows_v.at[sl], sem))
        copies.append(pltpu.async_copy(xyz_hbm.at[idx_v.at[sl]],
                                       xrows_v.at[sl], sem))
    for cp in copies:
        cp.wait()
    pltpu.sync_copy(rows_v, feats_out.at[pl.ds(base, _RPW)])
    pltpu.sync_copy(xrows_v, xyz_out.at[pl.ds(base, _RPW)])


def _sc_gather(nf_flat, xyz_flat, gidx):
    mesh = plsc.VectorSubcoreMesh(core_axis_name="c", subcore_axis_name="s")
    fn = functools.partial(
        pl.kernel, mesh=mesh,
        out_type=(jax.ShapeDtypeStruct((B * K, OUTC), jnp.float32),
                  jax.ShapeDtypeStruct((B * K, 128), jnp.float32)),
        scratch_types=[
            pltpu.VMEM((_RPW,), jnp.int32),
            pltpu.VMEM((_RPW, OUTC), jnp.float32),
            pltpu.VMEM((_RPW, 128), jnp.float32),
            pltpu.SemaphoreType.DMA,
        ],
    )(_sc_gather_body)
    return fn(nf_flat, xyz_flat, gidx)


def kernel(xyzs, features, gamma, beta, W1, b1, W2, b2, W3, b3, W4, b4):
    f = jnp.transpose(features, (0, 2, 1))
    mu = jnp.mean(f, axis=-1, keepdims=True)
    var = jnp.var(f, axis=-1, keepdims=True)
    f = (f - mu) / jnp.sqrt(var + 1e-6) * gamma + beta
    g = jax.nn.relu(f @ W3 + b3)
    w = (g @ W4 + b4)[:, :, 0]
    y = w / 0.1

    nf, xyz_pad = _nf_pallas(features, xyzs, gamma, beta, W1, b1, W2, b2)
    indices, gidx = _sort_pallas(y)

    nf_flat = nf.reshape(B * N, OUTC)
    xyz_flat = xyz_pad.reshape(B * N, 128)

    feats_g, xyz_g = _sc_gather(nf_flat, xyz_flat, gidx.reshape(-1))
    feats_out = jnp.transpose(feats_g.reshape(B, K, OUTC), (0, 2, 1))
    xyzs_out = xyz_g.reshape(B, K, 128)[:, :, :3]
    return (xyzs_out, feats_out, indices)
